# baseline (speedup 1.0000x reference)
import functools

import jax
import jax.numpy as jnp
from jax import lax
from jax.experimental import pallas as pl
from jax.experimental.pallas import tpu as pltpu
from jax.experimental.pallas import tpu_sc as plsc

N = 10000
E = 320000
D = 128
EPS = 1e-5

NC = 2
NS = 16
NW = NC * NS
EPW = E // NW
CHUNK = 80
NCHUNK = EPW // CHUNK
RPT = 632
N_PAD = NS * RPT
CW = 128


def _mesh():
    return plsc.VectorSubcoreMesh(core_axis_name="c", subcore_axis_name="s",
                                  num_cores=NC, num_subcores=NS)


def _sc_agg_body(with_count, *refs):
    if with_count:
        (table, packed3, zrow, ones_h, out, cnt_out, packed_all,
         sa_a, da_a, sa_b, da_b, rows_a, rows_b, acc,
         sem_a, sem_b, sem_sa, sem_sb) = refs
    else:
        (table, packed3, zrow, out, packed_all,
         sa_a, da_a, sa_b, da_b, rows_a, rows_b, acc,
         sem_a, sem_b, sem_sa, sem_sb) = refs

    c = lax.axis_index("c")
    s = lax.axis_index("s")
    wid = s * NC + c

    r0 = s * RPT
    pltpu.sync_copy(zrow.at[pl.ds(r0, RPT)], acc.at[pl.ds(r0, RPT)])

    pltpu.sync_copy(packed3.at[wid], packed_all)

    def unpack(i, sa, da):
        for k in range(CHUNK // 16):
            w = packed_all[i, pl.ds(k * 16, 16)]
            sa[pl.ds(k * 16, 16)] = w & 0xFFFF
            da[pl.ds(k * 16, 16)] = jnp.right_shift(w, 16)

    def wait_gather(sa, buf, sem):
        pltpu.make_async_copy(table.at[sa], buf, sem).wait()

    def scat(buf, da, sem):
        pltpu.async_copy(buf, acc.at[da], sem, add=True)

    def wait_scat(buf, da, sem):
        pltpu.make_async_copy(buf, acc.at[da], sem).wait()

    if with_count:
        pltpu.sync_copy(ones_h, rows_a)
        plsc.subcore_barrier()

        unpack(0, sa_a, da_a)
        scat(rows_a, da_a, sem_sa)

        def cpair(j, carry):
            i0 = 2 * j
            unpack(i0 + 1, sa_b, da_b)
            scat(rows_a, da_b, sem_sb)
            wait_scat(rows_a, da_a, sem_sa)
            unpack(i0 + 2, sa_a, da_a)
            scat(rows_a, da_a, sem_sa)
            wait_scat(rows_a, da_b, sem_sb)
            return carry

        lax.fori_loop(0, (NCHUNK - 3) // 2, cpair, 0)
        unpack(NCHUNK - 2, sa_b, da_b)
        scat(rows_a, da_b, sem_sb)
        wait_scat(rows_a, da_a, sem_sa)
        unpack(NCHUNK - 1, sa_a, da_a)
        scat(rows_a, da_a, sem_sa)
        wait_scat(rows_a, da_b, sem_sb)
        wait_scat(rows_a, da_a, sem_sa)

        plsc.subcore_barrier()
        pltpu.sync_copy(acc.at[pl.ds(r0, RPT)], cnt_out.at[c, pl.ds(r0, RPT)])
        pltpu.sync_copy(zrow.at[pl.ds(r0, RPT)], acc.at[pl.ds(r0, RPT)])

    plsc.subcore_barrier()

    unpack(0, sa_a, da_a)
    pltpu.async_copy(table.at[sa_a], rows_a, sem_a)
    unpack(1, sa_b, da_b)
    pltpu.async_copy(table.at[sa_b], rows_b, sem_b)

    def pair(j, carry):
        i0 = 2 * j
        wait_gather(sa_a, rows_a, sem_a)
        scat(rows_a, da_a, sem_sa)
        wait_gather(sa_b, rows_b, sem_b)
        scat(rows_b, da_b, sem_sb)
        wait_scat(rows_a, da_a, sem_sa)
        unpack(i0 + 2, sa_a, da_a)
        pltpu.async_copy(table.at[sa_a], rows_a, sem_a)
        wait_scat(rows_b, da_b, sem_sb)
        unpack(i0 + 3, sa_b, da_b)
        pltpu.async_copy(table.at[sa_b], rows_b, sem_b)
        return carry

    lax.fori_loop(0, (NCHUNK - 3) // 2, pair, 0)
    wait_gather(sa_a, rows_a, sem_a)
    scat(rows_a, da_a, sem_sa)
    wait_gather(sa_b, rows_b, sem_b)
    scat(rows_b, da_b, sem_sb)
    wait_scat(rows_a, da_a, sem_sa)
    unpack(NCHUNK - 1, sa_a, da_a)
    pltpu.async_copy(table.at[sa_a], rows_a, sem_a)
    wait_gather(sa_a, rows_a, sem_a)
    scat(rows_a, da_a, sem_sa)
    wait_scat(rows_b, da_b, sem_sb)
    wait_scat(rows_a, da_a, sem_sa)

    plsc.subcore_barrier()

    pltpu.sync_copy(acc.at[pl.ds(r0, RPT)], out.at[c, pl.ds(r0, RPT)])


@functools.lru_cache(maxsize=None)
def _sc_agg(with_count):
    if with_count:
        out_type = (jax.ShapeDtypeStruct((NC, N_PAD, D), jnp.float32),
                    jax.ShapeDtypeStruct((NC, N_PAD, CW), jnp.float32))
    else:
        out_type = jax.ShapeDtypeStruct((NC, N_PAD, D), jnp.float32)
    return pl.kernel(
        functools.partial(_sc_agg_body, with_count),
        out_type=out_type,
        mesh=_mesh(),
        scratch_types=(
            pltpu.VMEM((NCHUNK, CHUNK), jnp.int32),
            pltpu.VMEM((CHUNK,), jnp.int32),
            pltpu.VMEM((CHUNK,), jnp.int32),
            pltpu.VMEM((CHUNK,), jnp.int32),
            pltpu.VMEM((CHUNK,), jnp.int32),
            pltpu.VMEM((CHUNK, D), jnp.float32),
            pltpu.VMEM((CHUNK, D), jnp.float32),
            pltpu.VMEM_SHARED((N_PAD, D), jnp.float32),
            pltpu.SemaphoreType.DMA,
            pltpu.SemaphoreType.DMA,
            pltpu.SemaphoreType.DMA,
            pltpu.SemaphoreType.DMA,
        ))


def _relu_body(x_ref, o_ref):
    o_ref[...] = jnp.maximum(x_ref[...], 0.0)


def _tc_relu(x):
    return pl.pallas_call(
        _relu_body,
        out_shape=jax.ShapeDtypeStruct((N, D), jnp.float32),
    )(x)


def _tc_layer_body(p_ref, cnt_ref, xin_ref, wl_ref, bl_ref, wr_ref,
                   g_ref, b_ref, out_ref):
    ssum = p_ref[0, :N] + p_ref[1, :N]
    cn = cnt_ref[0, :N] + cnt_ref[1, :N]
    inv = 1.0 / jnp.maximum(cn[:, 0:1], 1.0)
    agg = ssum * inv
    t = (jnp.dot(agg, wl_ref[...], preferred_element_type=jnp.float32)
         + jnp.dot(xin_ref[...], wr_ref[...], preferred_element_type=jnp.float32)
         + bl_ref[...][None, :])
    mean = jnp.mean(t, axis=0)
    var = jnp.mean((t - mean[None, :]) ** 2, axis=0)
    h = (g_ref[...][None, :] * (t - mean[None, :]) * lax.rsqrt(var + EPS)
         + b_ref[...][None, :])
    out_ref[...] = jnp.maximum(h, 0.0)


def _tc_layer(p, cnt, xin, wl, bl, wr, g, b):
    return pl.pallas_call(
        _tc_layer_body,
        out_shape=jax.ShapeDtypeStruct((N, D), jnp.float32),
    )(p, cnt, xin, wl, bl, wr, g, b)


def kernel(x, edge_index, Wl1, bl1, Wr1, g1, b1, Wl2, bl2, Wr2, g2, b2):
    src = edge_index[0].astype(jnp.int32)
    dst = edge_index[1].astype(jnp.int32)
    packed3 = (src | (dst << 16)).reshape(NW, NCHUNK, CHUNK)
    zrow = jnp.zeros((N_PAD, D), jnp.float32)
    ones_h = jnp.ones((CHUNK, CW), jnp.float32)

    r1 = _tc_relu(x)
    p1, cnt = _sc_agg(True)(r1, packed3, zrow, ones_h)
    h1 = _tc_layer(p1, cnt, x, Wl1, bl1, Wr1, g1, b1)
    p2 = _sc_agg(False)(h1, packed3, zrow)
    out = _tc_layer(p2, cnt, h1, Wl2, bl2, Wr2, g2, b2)
    return out


# speedup vs baseline: 1.1796x; 1.1796x over previous
import functools

import jax
import jax.numpy as jnp
from jax import lax
from jax.experimental import pallas as pl
from jax.experimental.pallas import tpu as pltpu
from jax.experimental.pallas import tpu_sc as plsc

N = 10000
E = 320000
D = 128
EPS = 1e-5

NC = 2
NS = 16
NW = NC * NS
EPW = E // NW
CHUNK = 80
NCHUNK = EPW // CHUNK
RPT = 632
N_PAD = NS * RPT
CW = 128


def _mesh():
    return plsc.VectorSubcoreMesh(core_axis_name="c", subcore_axis_name="s",
                                  num_cores=NC, num_subcores=NS)


def _sc_agg_body(with_count, *refs):
    if with_count:
        (table, packed3, zrow, ones_h, out, cnt_out, packed_all,
         sa_a, da_a, sa_b, da_b, rows_a, rows_b, acc,
         sem_a, sem_b, sem_sa, sem_sb) = refs
    else:
        (table, packed3, zrow, out, packed_all,
         sa_a, da_a, sa_b, da_b, rows_a, rows_b, acc,
         sem_a, sem_b, sem_sa, sem_sb) = refs

    c = lax.axis_index("c")
    s = lax.axis_index("s")
    wid = s * NC + c

    r0 = s * RPT
    pltpu.sync_copy(zrow.at[pl.ds(r0, RPT)], acc.at[pl.ds(r0, RPT)])

    pltpu.sync_copy(packed3.at[wid], packed_all)

    def unpack(i, sa, da):
        for k in range(CHUNK // 16):
            w = packed_all[i, pl.ds(k * 16, 16)]
            sa[pl.ds(k * 16, 16)] = w & 0xFFFF
            da[pl.ds(k * 16, 16)] = jnp.right_shift(w, 16)

    def wait_gather(sa, buf, sem):
        pltpu.make_async_copy(table.at[sa], buf, sem).wait()

    def scat(buf, da, sem):
        pltpu.async_copy(buf, acc.at[da], sem, add=True)

    def wait_scat(buf, da, sem):
        pltpu.make_async_copy(buf, acc.at[da], sem).wait()

    if with_count:
        pltpu.sync_copy(ones_h, rows_a)
        plsc.subcore_barrier()

        def cstep(i, carry):
            unpack(i, sa_a, da_a)
            pltpu.sync_copy(rows_a, acc.at[da_a], add=True)
            return carry

        lax.fori_loop(0, NCHUNK, cstep, 0)
        plsc.subcore_barrier()
        pltpu.sync_copy(acc.at[pl.ds(r0, RPT)], cnt_out.at[c, pl.ds(r0, RPT)])
        pltpu.sync_copy(zrow.at[pl.ds(r0, RPT)], acc.at[pl.ds(r0, RPT)])

    plsc.subcore_barrier()

    unpack(0, sa_a, da_a)
    pltpu.async_copy(table.at[sa_a], rows_a, sem_a)

    def pair(j, carry):
        i0 = 2 * j
        unpack(i0 + 1, sa_b, da_b)
        pltpu.async_copy(table.at[sa_b], rows_b, sem_b)
        wait_gather(sa_a, rows_a, sem_a)
        pltpu.sync_copy(rows_a, acc.at[da_a], add=True)
        unpack(i0 + 2, sa_a, da_a)
        pltpu.async_copy(table.at[sa_a], rows_a, sem_a)
        wait_gather(sa_b, rows_b, sem_b)
        pltpu.sync_copy(rows_b, acc.at[da_b], add=True)
        return carry

    lax.fori_loop(0, (NCHUNK - 1) // 2, pair, 0)
    wait_gather(sa_a, rows_a, sem_a)
    pltpu.sync_copy(rows_a, acc.at[da_a], add=True)

    plsc.subcore_barrier()

    pltpu.sync_copy(acc.at[pl.ds(r0, RPT)], out.at[c, pl.ds(r0, RPT)])


@functools.lru_cache(maxsize=None)
def _sc_agg(with_count):
    if with_count:
        out_type = (jax.ShapeDtypeStruct((NC, N_PAD, D), jnp.float32),
                    jax.ShapeDtypeStruct((NC, N_PAD, CW), jnp.float32))
    else:
        out_type = jax.ShapeDtypeStruct((NC, N_PAD, D), jnp.float32)
    return pl.kernel(
        functools.partial(_sc_agg_body, with_count),
        out_type=out_type,
        mesh=_mesh(),
        scratch_types=(
            pltpu.VMEM((NCHUNK, CHUNK), jnp.int32),
            pltpu.VMEM((CHUNK,), jnp.int32),
            pltpu.VMEM((CHUNK,), jnp.int32),
            pltpu.VMEM((CHUNK,), jnp.int32),
            pltpu.VMEM((CHUNK,), jnp.int32),
            pltpu.VMEM((CHUNK, D), jnp.float32),
            pltpu.VMEM((CHUNK, D), jnp.float32),
            pltpu.VMEM_SHARED((N_PAD, D), jnp.float32),
            pltpu.SemaphoreType.DMA,
            pltpu.SemaphoreType.DMA,
            pltpu.SemaphoreType.DMA,
            pltpu.SemaphoreType.DMA,
        ))


def _relu_body(x_ref, o_ref):
    o_ref[...] = jnp.maximum(x_ref[...], 0.0)


def _tc_relu(x):
    return pl.pallas_call(
        _relu_body,
        out_shape=jax.ShapeDtypeStruct((N, D), jnp.float32),
    )(x)


def _tc_layer_body(p_ref, cnt_ref, xin_ref, wl_ref, bl_ref, wr_ref,
                   g_ref, b_ref, out_ref):
    ssum = p_ref[0, :N] + p_ref[1, :N]
    cn = cnt_ref[0, :N] + cnt_ref[1, :N]
    inv = 1.0 / jnp.maximum(cn[:, 0:1], 1.0)
    agg = ssum * inv
    t = (jnp.dot(agg, wl_ref[...], preferred_element_type=jnp.float32)
         + jnp.dot(xin_ref[...], wr_ref[...], preferred_element_type=jnp.float32)
         + bl_ref[...][None, :])
    mean = jnp.mean(t, axis=0)
    var = jnp.mean((t - mean[None, :]) ** 2, axis=0)
    h = (g_ref[...][None, :] * (t - mean[None, :]) * lax.rsqrt(var + EPS)
         + b_ref[...][None, :])
    out_ref[...] = jnp.maximum(h, 0.0)


def _tc_layer(p, cnt, xin, wl, bl, wr, g, b):
    return pl.pallas_call(
        _tc_layer_body,
        out_shape=jax.ShapeDtypeStruct((N, D), jnp.float32),
    )(p, cnt, xin, wl, bl, wr, g, b)


def kernel(x, edge_index, Wl1, bl1, Wr1, g1, b1, Wl2, bl2, Wr2, g2, b2):
    src = edge_index[0].astype(jnp.int32)
    dst = edge_index[1].astype(jnp.int32)
    packed3 = (src | (dst << 16)).reshape(NW, NCHUNK, CHUNK)
    zrow = jnp.zeros((N_PAD, D), jnp.float32)
    ones_h = jnp.ones((CHUNK, CW), jnp.float32)

    r1 = _tc_relu(x)
    p1, cnt = _sc_agg(True)(r1, packed3, zrow, ones_h)
    h1 = _tc_layer(p1, cnt, x, Wl1, bl1, Wr1, g1, b1)
    p2 = _sc_agg(False)(h1, packed3, zrow)
    out = _tc_layer(p2, cnt, h1, Wl2, bl2, Wr2, g2, b2)
    return out
